# Initial kernel scaffold; baseline (speedup 1.0000x reference)
import jax
import jax.numpy as jnp
from jax.experimental import pallas as pl


def kernel(key, hidden_state, visible_state, logpsi, linear_conf, n, W_h, W_v, b_h, b_v, W_s):
    raise NotImplementedError("write your pallas kernel here")


# speedup vs baseline: 4.8556x; 4.8556x over previous
import functools
from itertools import product

import numpy as np
import jax
import jax.numpy as jnp
from jax import lax
from jax.experimental import pallas as pl
from jax.experimental.pallas import tpu as pltpu
from jax.experimental.pallas import tpu_sc as plsc

_tetra = np.array(
    [[0.0, 0.0, 1.0],
     [(8.0 / 9.0) ** 0.5, 0.0, -1.0 / 3.0],
     [-(2.0 / 9.0) ** 0.5, (2.0 / 3.0) ** 0.5, -1.0 / 3.0],
     [-(2.0 / 9.0) ** 0.5, -(2.0 / 3.0) ** 0.5, -1.0 / 3.0]],
    dtype=np.float32)
_confs = np.array(list(product(range(4), repeat=4)), dtype=np.int32)
_vecs = _tetra[_confs]
_VCONST = np.concatenate([
    _vecs.transpose(1, 2, 0).reshape(-1),
    _vecs.reshape(-1),
])

_WH, _WV, _BH, _BV, _LP, _CONF = 0, 48, 96, 108, 120, 136

_GDN = lax.GatherDimensionNumbers(
    offset_dims=(), collapsed_slice_dims=(0,), start_index_map=(0,))


def _vtake(x, idx):
    return lax.gather(x, idx[:, None], _GDN, (1,),
                      mode=lax.GatherScatterMode.PROMISE_IN_BOUNDS)


def _splat_i(v):
    return lax.broadcast(jnp.asarray(v, jnp.int32), (16,))


def _rsqrt(x):
    b = plsc.bitcast(x, jnp.int32)
    y = plsc.bitcast(jnp.int32(0x5F3759DF) - (b >> 1), jnp.float32)
    for _ in range(3):
        y = y * (1.5 - 0.5 * x * y * y)
    return y


def _vlog(x):
    b = plsc.bitcast(x, jnp.int32)
    e = ((b >> 23) & 0xFF) - 127
    m = plsc.bitcast((b & 0x007FFFFF) | 0x3F800000, jnp.float32)
    big = m > 1.4142135
    m = jnp.where(big, 0.5 * m, m)
    e = e + jnp.where(big, 1, 0)
    t = (m - 1.0) / (m + 1.0)
    t2 = t * t
    p = 2.0 * t * (1.0 + t2 * (1.0 / 3.0 + t2 * (0.2 + t2 * (1.0 / 7.0 + t2 * (1.0 / 9.0)))))
    return e.astype(jnp.float32) * 0.6931471805599453 + p


def _body(hid_hbm, vis_hbm, wb_hbm, ws_hbm, vc_hbm,
          o_hid, o_vis, o_lp,
          hidv, visv, wbv, wsv, vcv, btab, ytab, wtab, lpv, sem):
    cid = lax.axis_index("c")
    sid = lax.axis_index("s")

    @pl.when(jnp.logical_and(cid == 0, sid == 0))
    def _():
        cps = [pltpu.async_copy(hid_hbm, hidv, sem),
               pltpu.async_copy(vis_hbm, visv, sem),
               pltpu.async_copy(wb_hbm, wbv, sem),
               pltpu.async_copy(ws_hbm, wsv, sem),
               pltpu.async_copy(vc_hbm, vcv, sem)]
        for c in cps:
            c.wait()

        lane = lax.iota(jnp.int32, 16)
        mrow = lane // 3
        vcomp = lane - 3 * mrow
        in12 = lane < 12
        rot1 = jnp.where(in12, 3 * mrow + (vcomp + 1) % 3, lane)
        rot2 = jnp.where(in12, 3 * mrow + (vcomp + 2) % 3, lane)
        safe_lane = jnp.where(in12, lane, 0)
        safe_row = jnp.where(in12, mrow, 0)

        for a in range(3):
            for j in range(4):
                wtab[pl.ds((a * 4 + j) * 16, 16)] = plsc.load_gather(
                    wbv, [_WH + a * 16 + safe_row * 4 + j])
                wtab[pl.ds((12 + a * 4 + j) * 16, 16)] = plsc.load_gather(
                    wbv, [_WV + a * 16 + safe_row * 4 + j])
            wtab[pl.ds((24 + a) * 16, 16)] = plsc.load_gather(
                wbv, [_BH + a * 4 + safe_row])
            wtab[pl.ds((27 + a) * 16, 16)] = plsc.load_gather(
                wbv, [_BV + a * 4 + safe_row])

        def bfold(i, carry):
            wregs = [plsc.load_gather(wsv, [i * 256 + lane * 16 + r])
                     for r in range(16)]
            vv = [vcv[pl.ds(r * 256 + i * 16, 16)] for r in range(12)]
            for h in range(4):
                for v in range(3):
                    acc = vv[v] * wregs[h]
                    for b in range(1, 4):
                        acc = acc + vv[b * 3 + v] * wregs[b * 4 + h]
                    btab[pl.ds((h * 3 + v) * 256 + i * 16, 16)] = acc
            return carry

        lax.fori_loop(0, 16, bfold, 0)

        def local_update(state_ref, wbase, bbase, t):
            tot = None
            for a, bit in enumerate((4, 2, 1)):
                nb = (t ^ bit) * 12
                prev = plsc.load_gather(state_ref, [nb + safe_lane])
                sq = prev * prev
                ln2 = sq + _vtake(sq, rot1) + _vtake(sq, rot2)
                inv = _rsqrt(ln2)
                lng = ln2 * inv
                acc = wtab[pl.ds((bbase + a) * 16, 16)]
                for j in range(4):
                    acc = acc + (wtab[pl.ds((wbase + a * 4 + j) * 16, 16)]
                                 * _vtake(lng, _splat_i(j * 3)))
                xe = jnp.where(acc > 0, acc, jnp.exp(acc) - 1.0)
                contrib = xe * prev * inv
                tot = contrib if tot is None else tot + contrib
            return tot

        lp = wbv[pl.ds(_LP, 16)]
        conf_f = wbv[pl.ds(_CONF, 16)]

        for t in range(8):
            loc = (local_update(hidv, 0, 24, t)
                   + local_update(visv, 12, 27, t))
            plsc.store_scatter(hidv, [t * 12 + safe_lane], loc, mask=in12)
            lb = [_vtake(loc, _splat_i(d)) for d in range(12)]

            def ybody(i, mv):
                acc = btab[pl.ds(i * 16, 16)] * lb[0]
                for d in range(1, 12):
                    acc = acc + btab[pl.ds(d * 256 + i * 16, 16)] * lb[d]
                ytab[pl.ds(i * 16, 16)] = acc
                return jnp.maximum(mv, acc)

            mvec = lax.fori_loop(0, 16, ybody,
                                 jnp.full((16,), -jnp.inf, jnp.float32))
            msp = lax.broadcast(jnp.max(mvec), (16,))

            def sbody(i, sv):
                return sv + jnp.exp(ytab[pl.ds(i * 16, 16)] - msp)

            svec = lax.fori_loop(0, 16, sbody, jnp.zeros((16,), jnp.float32))
            logs = _vlog(lax.broadcast(jnp.sum(svec), (16,)))

            cvec = plsc.bitcast(_vtake(conf_f, _splat_i(t)), jnp.int32)
            ycv = plsc.load_gather(ytab, [cvec])
            lp = lp + 0.5 * (ycv - msp - logs)

            vis_new = plsc.load_gather(vcv, [3072 + cvec * 12 + safe_lane])
            plsc.store_scatter(visv, [t * 12 + safe_lane], vis_new, mask=in12)

        lpv[pl.ds(0, 16)] = lp
        pltpu.sync_copy(hidv, o_hid)
        pltpu.sync_copy(visv, o_vis)
        pltpu.sync_copy(lpv, o_lp)


@functools.cache
def _sc_call():
    return pl.kernel(
        _body,
        out_type=(jax.ShapeDtypeStruct((96,), jnp.float32),
                  jax.ShapeDtypeStruct((96,), jnp.float32),
                  jax.ShapeDtypeStruct((16,), jnp.float32)),
        mesh=plsc.VectorSubcoreMesh(core_axis_name="c", subcore_axis_name="s"),
        compiler_params=pltpu.CompilerParams(needs_layout_passes=False),
        scratch_types=[
            pltpu.VMEM((96,), jnp.float32),
            pltpu.VMEM((96,), jnp.float32),
            pltpu.VMEM((160,), jnp.float32),
            pltpu.VMEM((4096,), jnp.float32),
            pltpu.VMEM((6144,), jnp.float32),
            pltpu.VMEM((3072,), jnp.float32),
            pltpu.VMEM((256,), jnp.float32),
            pltpu.VMEM((480,), jnp.float32),
            pltpu.VMEM((16,), jnp.float32),
            pltpu.SemaphoreType.DMA,
        ],
    )


def kernel(key, hidden_state, visible_state, logpsi, linear_conf, n,
           W_h, W_v, b_h, b_v, W_s):
    del key, n
    wb = jnp.concatenate([
        W_h.reshape(48), W_v.reshape(48), b_h.reshape(12), b_v.reshape(12),
        logpsi.reshape(1), jnp.zeros((15,), jnp.float32),
        lax.bitcast_convert_type(linear_conf, jnp.float32),
        jnp.zeros((16,), jnp.float32),
    ])
    hid_f, vis_f, lp16 = _sc_call()(
        hidden_state.reshape(96), visible_state.reshape(96), wb,
        W_s.reshape(4096), jnp.asarray(_VCONST))
    return (hid_f.reshape(2, 2, 2, 4, 3), vis_f.reshape(2, 2, 2, 4, 3),
            lp16[0], linear_conf)


# speedup vs baseline: 4.8733x; 1.0037x over previous
import functools
from itertools import product

import numpy as np
import jax
import jax.numpy as jnp
from jax import lax
from jax.experimental import pallas as pl
from jax.experimental.pallas import tpu as pltpu
from jax.experimental.pallas import tpu_sc as plsc

_tetra = np.array(
    [[0.0, 0.0, 1.0],
     [(8.0 / 9.0) ** 0.5, 0.0, -1.0 / 3.0],
     [-(2.0 / 9.0) ** 0.5, (2.0 / 3.0) ** 0.5, -1.0 / 3.0],
     [-(2.0 / 9.0) ** 0.5, -(2.0 / 3.0) ** 0.5, -1.0 / 3.0]],
    dtype=np.float32)
_confs = np.array(list(product(range(4), repeat=4)), dtype=np.int32)
_vecs = _tetra[_confs]
_VCONST = np.concatenate([
    _vecs.transpose(1, 2, 0).reshape(-1),
    _vecs.reshape(-1),
])

_WH, _WV, _BH, _BV, _LP, _CONF = 0, 48, 96, 108, 120, 136

_GDN = lax.GatherDimensionNumbers(
    offset_dims=(), collapsed_slice_dims=(0,), start_index_map=(0,))


def _vtake(x, idx):
    return lax.gather(x, idx[:, None], _GDN, (1,),
                      mode=lax.GatherScatterMode.PROMISE_IN_BOUNDS)


def _splat_i(v):
    return lax.broadcast(jnp.asarray(v, jnp.int32), (16,))


def _rsqrt(x):
    b = plsc.bitcast(x, jnp.int32)
    y = plsc.bitcast(jnp.int32(0x5F3759DF) - (b >> 1), jnp.float32)
    for _ in range(3):
        y = y * (1.5 - 0.5 * x * y * y)
    return y


def _vlog(x):
    b = plsc.bitcast(x, jnp.int32)
    e = ((b >> 23) & 0xFF) - 127
    m = plsc.bitcast((b & 0x007FFFFF) | 0x3F800000, jnp.float32)
    big = m > 1.4142135
    m = jnp.where(big, 0.5 * m, m)
    e = e + jnp.where(big, 1, 0)
    t = (m - 1.0) / (m + 1.0)
    t2 = t * t
    p = 2.0 * t * (1.0 + t2 * (1.0 / 3.0 + t2 * (0.2 + t2 * (1.0 / 7.0 + t2 * (1.0 / 9.0)))))
    return e.astype(jnp.float32) * 0.6931471805599453 + p


def _body(hid_hbm, vis_hbm, wb_hbm, ws_hbm, vc_hbm,
          o_hid, o_vis, o_lp,
          hidv, visv, wbv, wsv, vcv, btab, ytab, wtab, lpv, sem):
    if True:
        cps = [pltpu.async_copy(hid_hbm, hidv, sem),
               pltpu.async_copy(vis_hbm, visv, sem),
               pltpu.async_copy(wb_hbm, wbv, sem),
               pltpu.async_copy(ws_hbm, wsv, sem),
               pltpu.async_copy(vc_hbm, vcv, sem)]
        for c in cps:
            c.wait()

        lane = lax.iota(jnp.int32, 16)
        mrow = lane // 3
        vcomp = lane - 3 * mrow
        in12 = lane < 12
        rot1 = jnp.where(in12, 3 * mrow + (vcomp + 1) % 3, lane)
        rot2 = jnp.where(in12, 3 * mrow + (vcomp + 2) % 3, lane)
        safe_lane = jnp.where(in12, lane, 0)
        safe_row = jnp.where(in12, mrow, 0)

        for a in range(3):
            for j in range(4):
                wtab[pl.ds((a * 4 + j) * 16, 16)] = plsc.load_gather(
                    wbv, [_WH + a * 16 + safe_row * 4 + j])
                wtab[pl.ds((12 + a * 4 + j) * 16, 16)] = plsc.load_gather(
                    wbv, [_WV + a * 16 + safe_row * 4 + j])
            wtab[pl.ds((24 + a) * 16, 16)] = plsc.load_gather(
                wbv, [_BH + a * 4 + safe_row])
            wtab[pl.ds((27 + a) * 16, 16)] = plsc.load_gather(
                wbv, [_BV + a * 4 + safe_row])

        def bfold(i, carry):
            wregs = [plsc.load_gather(wsv, [i * 256 + lane * 16 + r])
                     for r in range(16)]
            vv = [vcv[pl.ds(r * 256 + i * 16, 16)] for r in range(12)]
            for h in range(4):
                for v in range(3):
                    acc = vv[v] * wregs[h]
                    for b in range(1, 4):
                        acc = acc + vv[b * 3 + v] * wregs[b * 4 + h]
                    btab[pl.ds((h * 3 + v) * 256 + i * 16, 16)] = acc
            return carry

        lax.fori_loop(0, 16, bfold, 0)

        def local_update(state_ref, wbase, bbase, t):
            tot = None
            for a, bit in enumerate((4, 2, 1)):
                nb = (t ^ bit) * 12
                prev = plsc.load_gather(state_ref, [nb + safe_lane])
                sq = prev * prev
                ln2 = sq + _vtake(sq, rot1) + _vtake(sq, rot2)
                inv = _rsqrt(ln2)
                lng = ln2 * inv
                acc = wtab[pl.ds((bbase + a) * 16, 16)]
                for j in range(4):
                    acc = acc + (wtab[pl.ds((wbase + a * 4 + j) * 16, 16)]
                                 * _vtake(lng, _splat_i(j * 3)))
                xe = jnp.where(acc > 0, acc, jnp.exp(acc) - 1.0)
                contrib = xe * prev * inv
                tot = contrib if tot is None else tot + contrib
            return tot

        lp = wbv[pl.ds(_LP, 16)]
        conf_f = wbv[pl.ds(_CONF, 16)]

        for t in range(8):
            loc = (local_update(hidv, 0, 24, t)
                   + local_update(visv, 12, 27, t))
            plsc.store_scatter(hidv, [t * 12 + safe_lane], loc, mask=in12)
            lb = [_vtake(loc, _splat_i(d)) for d in range(12)]

            def ybody(o, mv):
                for q in range(4):
                    i = o * 4 + q
                    acc = btab[pl.ds(i * 16, 16)] * lb[0]
                    for d in range(1, 12):
                        acc = acc + btab[pl.ds(d * 256 + i * 16, 16)] * lb[d]
                    ytab[pl.ds(i * 16, 16)] = acc
                    mv = jnp.maximum(mv, acc)
                return mv

            mvec = lax.fori_loop(0, 4, ybody,
                                 jnp.full((16,), -jnp.inf, jnp.float32))
            msp = lax.broadcast(jnp.max(mvec), (16,))

            def sbody(o, sv):
                for q in range(4):
                    sv = sv + jnp.exp(ytab[pl.ds((o * 4 + q) * 16, 16)] - msp)
                return sv

            svec = lax.fori_loop(0, 4, sbody, jnp.zeros((16,), jnp.float32))
            logs = _vlog(lax.broadcast(jnp.sum(svec), (16,)))

            cvec = plsc.bitcast(_vtake(conf_f, _splat_i(t)), jnp.int32)
            ycv = plsc.load_gather(ytab, [cvec])
            lp = lp + 0.5 * (ycv - msp - logs)

            vis_new = plsc.load_gather(vcv, [3072 + cvec * 12 + safe_lane])
            plsc.store_scatter(visv, [t * 12 + safe_lane], vis_new, mask=in12)

        lpv[pl.ds(0, 16)] = lp
        pltpu.sync_copy(hidv, o_hid)
        pltpu.sync_copy(visv, o_vis)
        pltpu.sync_copy(lpv, o_lp)


@functools.cache
def _sc_call():
    return pl.kernel(
        _body,
        out_type=(jax.ShapeDtypeStruct((96,), jnp.float32),
                  jax.ShapeDtypeStruct((96,), jnp.float32),
                  jax.ShapeDtypeStruct((16,), jnp.float32)),
        mesh=plsc.VectorSubcoreMesh(core_axis_name="c", subcore_axis_name="s",
                                    num_cores=1, num_subcores=1),
        compiler_params=pltpu.CompilerParams(needs_layout_passes=False),
        scratch_types=[
            pltpu.VMEM((96,), jnp.float32),
            pltpu.VMEM((96,), jnp.float32),
            pltpu.VMEM((160,), jnp.float32),
            pltpu.VMEM((4096,), jnp.float32),
            pltpu.VMEM((6144,), jnp.float32),
            pltpu.VMEM((3072,), jnp.float32),
            pltpu.VMEM((256,), jnp.float32),
            pltpu.VMEM((480,), jnp.float32),
            pltpu.VMEM((16,), jnp.float32),
            pltpu.SemaphoreType.DMA,
        ],
    )


def kernel(key, hidden_state, visible_state, logpsi, linear_conf, n,
           W_h, W_v, b_h, b_v, W_s):
    del key, n
    wb = jnp.concatenate([
        W_h.reshape(48), W_v.reshape(48), b_h.reshape(12), b_v.reshape(12),
        logpsi.reshape(1), jnp.zeros((15,), jnp.float32),
        lax.bitcast_convert_type(linear_conf, jnp.float32),
        jnp.zeros((16,), jnp.float32),
    ])
    hid_f, vis_f, lp16 = _sc_call()(
        hidden_state.reshape(96), visible_state.reshape(96), wb,
        W_s.reshape(4096), jnp.asarray(_VCONST))
    return (hid_f.reshape(2, 2, 2, 4, 3), vis_f.reshape(2, 2, 2, 4, 3),
            lp16[0], linear_conf)


# speedup vs baseline: 5.4257x; 1.1133x over previous
import functools
from itertools import product

import numpy as np
import jax
import jax.numpy as jnp
from jax import lax
from jax.experimental import pallas as pl
from jax.experimental.pallas import tpu as pltpu
from jax.experimental.pallas import tpu_sc as plsc

_tetra = np.array(
    [[0.0, 0.0, 1.0],
     [(8.0 / 9.0) ** 0.5, 0.0, -1.0 / 3.0],
     [-(2.0 / 9.0) ** 0.5, (2.0 / 3.0) ** 0.5, -1.0 / 3.0],
     [-(2.0 / 9.0) ** 0.5, -(2.0 / 3.0) ** 0.5, -1.0 / 3.0]],
    dtype=np.float32)
_confs = np.array(list(product(range(4), repeat=4)), dtype=np.int32)
_vecs = _tetra[_confs]
_VCONST = np.concatenate([
    _vecs.transpose(1, 2, 0).reshape(-1),
    _vecs.reshape(-1),
])

_WH, _WV, _BH, _BV, _LP, _CONF = 0, 48, 96, 108, 120, 136

_GDN = lax.GatherDimensionNumbers(
    offset_dims=(), collapsed_slice_dims=(0,), start_index_map=(0,))


def _vtake(x, idx):
    return lax.gather(x, idx[:, None], _GDN, (1,),
                      mode=lax.GatherScatterMode.PROMISE_IN_BOUNDS)


def _splat_i(v):
    return lax.broadcast(jnp.asarray(v, jnp.int32), (16,))


def _rsqrt(x):
    b = plsc.bitcast(x, jnp.int32)
    y = plsc.bitcast(jnp.int32(0x5F3759DF) - (b >> 1), jnp.float32)
    for _ in range(3):
        y = y * (1.5 - 0.5 * x * y * y)
    return y


def _vlog(x):
    b = plsc.bitcast(x, jnp.int32)
    e = ((b >> 23) & 0xFF) - 127
    m = plsc.bitcast((b & 0x007FFFFF) | 0x3F800000, jnp.float32)
    big = m > 1.4142135
    m = jnp.where(big, 0.5 * m, m)
    e = e + jnp.where(big, 1, 0)
    t = (m - 1.0) / (m + 1.0)
    t2 = t * t
    p = 2.0 * t * (1.0 + t2 * (1.0 / 3.0 + t2 * (0.2 + t2 * (1.0 / 7.0 + t2 * (1.0 / 9.0)))))
    return e.astype(jnp.float32) * 0.6931471805599453 + p


def _body(hid_hbm, vis_hbm, wb_hbm, ws_hbm, vc_hbm,
          o_hid, o_vis, o_lp,
          hidv, visv, wbv, wsv, vcv, btab, ytab, wtab, lpv, sem):
    if True:
        cps = [pltpu.async_copy(hid_hbm, hidv, sem),
               pltpu.async_copy(vis_hbm, visv, sem),
               pltpu.async_copy(wb_hbm, wbv, sem),
               pltpu.async_copy(ws_hbm, wsv, sem),
               pltpu.async_copy(vc_hbm, vcv, sem)]
        for c in cps:
            c.wait()

        lane = lax.iota(jnp.int32, 16)
        mrow = lane // 3
        vcomp = lane - 3 * mrow
        in12 = lane < 12
        rot1 = jnp.where(in12, 3 * mrow + (vcomp + 1) % 3, lane)
        rot2 = jnp.where(in12, 3 * mrow + (vcomp + 2) % 3, lane)
        safe_lane = jnp.where(in12, lane, 0)
        safe_row = jnp.where(in12, mrow, 0)

        for a in range(3):
            for j in range(4):
                wtab[pl.ds((a * 4 + j) * 16, 16)] = plsc.load_gather(
                    wbv, [_WH + a * 16 + safe_row * 4 + j])
                wtab[pl.ds((12 + a * 4 + j) * 16, 16)] = plsc.load_gather(
                    wbv, [_WV + a * 16 + safe_row * 4 + j])
            wtab[pl.ds((24 + a) * 16, 16)] = plsc.load_gather(
                wbv, [_BH + a * 4 + safe_row])
            wtab[pl.ds((27 + a) * 16, 16)] = plsc.load_gather(
                wbv, [_BV + a * 4 + safe_row])

        def bfold(i, carry):
            wregs = [plsc.load_gather(wsv, [i * 256 + lane * 16 + r])
                     for r in range(16)]
            vv = [vcv[pl.ds(r * 256 + i * 16, 16)] for r in range(12)]
            for h in range(4):
                for v in range(3):
                    acc = vv[v] * wregs[h]
                    for b in range(1, 4):
                        acc = acc + vv[b * 3 + v] * wregs[b * 4 + h]
                    btab[pl.ds((h * 3 + v) * 256 + i * 16, 16)] = acc
            return carry

        lax.fori_loop(0, 16, bfold, 0)

        def local_update(state_ref, wbase, bbase, t):
            tot = None
            for a, bit in enumerate((4, 2, 1)):
                nb = (t ^ bit) * 12
                prev = plsc.load_gather(state_ref, [nb + safe_lane])
                sq = prev * prev
                ln2 = sq + _vtake(sq, rot1) + _vtake(sq, rot2)
                inv = _rsqrt(ln2)
                lng = ln2 * inv
                acc = wtab[pl.ds((bbase + a) * 16, 16)]
                for j in range(4):
                    acc = acc + (wtab[pl.ds((wbase + a * 4 + j) * 16, 16)]
                                 * _vtake(lng, _splat_i(j * 3)))
                xe = jnp.where(acc > 0, acc, jnp.exp(acc) - 1.0)
                contrib = xe * prev * inv
                tot = contrib if tot is None else tot + contrib
            return tot

        lp = wbv[pl.ds(_LP, 16)]
        conf_f = wbv[pl.ds(_CONF, 16)]

        def local_update_t(state_ref, wbase, bbase, t):
            tot = None
            for a, bit in enumerate((4, 2, 1)):
                nb = (t ^ bit) * 12
                prev = plsc.load_gather(state_ref, [nb + safe_lane])
                sq = prev * prev
                ln2 = sq + _vtake(sq, rot1) + _vtake(sq, rot2)
                inv = _rsqrt(ln2)
                lng = ln2 * inv
                acc = wtab[pl.ds((bbase + a) * 16, 16)]
                for j in range(4):
                    acc = acc + (wtab[pl.ds((wbase + a * 4 + j) * 16, 16)]
                                 * _vtake(lng, _splat_i(j * 3)))
                xe = jnp.where(acc > 0, acc, jnp.exp(acc) - 1.0)
                contrib = xe * prev * inv
                tot = contrib if tot is None else tot + contrib
            return tot

        def step(t, lp):
            loc = (local_update_t(hidv, 0, 24, t)
                   + local_update_t(visv, 12, 27, t))
            plsc.store_scatter(hidv, [t * 12 + safe_lane], loc, mask=in12)
            lb = [_vtake(loc, _splat_i(d)) for d in range(12)]

            def ybody(i, mv):
                acc = btab[pl.ds(i * 16, 16)] * lb[0]
                for d in range(1, 12):
                    acc = acc + btab[pl.ds(d * 256 + i * 16, 16)] * lb[d]
                ytab[pl.ds(i * 16, 16)] = acc
                return jnp.maximum(mv, acc)

            mvec = lax.fori_loop(0, 16, ybody,
                                 jnp.full((16,), -jnp.inf, jnp.float32))
            msp = lax.broadcast(jnp.max(mvec), (16,))

            def sbody(i, sv):
                return sv + jnp.exp(ytab[pl.ds(i * 16, 16)] - msp)

            svec = lax.fori_loop(0, 16, sbody, jnp.zeros((16,), jnp.float32))
            logs = _vlog(lax.broadcast(jnp.sum(svec), (16,)))

            cvec = plsc.bitcast(_vtake(conf_f, lax.broadcast(t, (16,))),
                                jnp.int32)
            ycv = plsc.load_gather(ytab, [cvec])
            lp = lp + 0.5 * (ycv - msp - logs)

            vis_new = plsc.load_gather(vcv, [3072 + cvec * 12 + safe_lane])
            plsc.store_scatter(visv, [t * 12 + safe_lane], vis_new, mask=in12)
            return lp

        lp = lax.fori_loop(0, 8, step, lp)

        lpv[pl.ds(0, 16)] = lp
        pltpu.sync_copy(hidv, o_hid)
        pltpu.sync_copy(visv, o_vis)
        pltpu.sync_copy(lpv, o_lp)


@functools.cache
def _sc_call():
    return pl.kernel(
        _body,
        out_type=(jax.ShapeDtypeStruct((96,), jnp.float32),
                  jax.ShapeDtypeStruct((96,), jnp.float32),
                  jax.ShapeDtypeStruct((16,), jnp.float32)),
        mesh=plsc.VectorSubcoreMesh(core_axis_name="c", subcore_axis_name="s",
                                    num_cores=1, num_subcores=1),
        compiler_params=pltpu.CompilerParams(needs_layout_passes=False),
        scratch_types=[
            pltpu.VMEM((96,), jnp.float32),
            pltpu.VMEM((96,), jnp.float32),
            pltpu.VMEM((160,), jnp.float32),
            pltpu.VMEM((4096,), jnp.float32),
            pltpu.VMEM((6144,), jnp.float32),
            pltpu.VMEM((3072,), jnp.float32),
            pltpu.VMEM((256,), jnp.float32),
            pltpu.VMEM((480,), jnp.float32),
            pltpu.VMEM((16,), jnp.float32),
            pltpu.SemaphoreType.DMA,
        ],
    )


def kernel(key, hidden_state, visible_state, logpsi, linear_conf, n,
           W_h, W_v, b_h, b_v, W_s):
    del key, n
    wb = jnp.concatenate([
        W_h.reshape(48), W_v.reshape(48), b_h.reshape(12), b_v.reshape(12),
        logpsi.reshape(1), jnp.zeros((15,), jnp.float32),
        lax.bitcast_convert_type(linear_conf, jnp.float32),
        jnp.zeros((16,), jnp.float32),
    ])
    hid_f, vis_f, lp16 = _sc_call()(
        hidden_state.reshape(96), visible_state.reshape(96), wb,
        W_s.reshape(4096), jnp.asarray(_VCONST))
    return (hid_f.reshape(2, 2, 2, 4, 3), vis_f.reshape(2, 2, 2, 4, 3),
            lp16[0], linear_conf)


# speedup vs baseline: 5.4644x; 1.0071x over previous
import functools
from itertools import product

import numpy as np
import jax
import jax.numpy as jnp
from jax import lax
from jax.experimental import pallas as pl
from jax.experimental.pallas import tpu as pltpu
from jax.experimental.pallas import tpu_sc as plsc

_tetra = np.array(
    [[0.0, 0.0, 1.0],
     [(8.0 / 9.0) ** 0.5, 0.0, -1.0 / 3.0],
     [-(2.0 / 9.0) ** 0.5, (2.0 / 3.0) ** 0.5, -1.0 / 3.0],
     [-(2.0 / 9.0) ** 0.5, -(2.0 / 3.0) ** 0.5, -1.0 / 3.0]],
    dtype=np.float32)
_confs = np.array(list(product(range(4), repeat=4)), dtype=np.int32)
_vecs = _tetra[_confs]
_VCONST = np.concatenate([
    _vecs.transpose(1, 2, 0).reshape(-1),
    _vecs.reshape(-1),
])

_HID, _VIS, _LP, _WH, _WV, _BH, _BV, _CONF, _WS = (
    0, 96, 192, 208, 256, 304, 316, 328, 344)
_PLEN = 344 + 4096

_GDN = lax.GatherDimensionNumbers(
    offset_dims=(), collapsed_slice_dims=(0,), start_index_map=(0,))


def _vtake(x, idx):
    return lax.gather(x, idx[:, None], _GDN, (1,),
                      mode=lax.GatherScatterMode.PROMISE_IN_BOUNDS)


def _splat_i(v):
    return lax.broadcast(jnp.asarray(v, jnp.int32), (16,))


def _rsqrt(x):
    b = plsc.bitcast(x, jnp.int32)
    y = plsc.bitcast(jnp.int32(0x5F3759DF) - (b >> 1), jnp.float32)
    for _ in range(3):
        y = y * (1.5 - 0.5 * x * y * y)
    return y


def _vlog(x):
    b = plsc.bitcast(x, jnp.int32)
    e = ((b >> 23) & 0xFF) - 127
    m = plsc.bitcast((b & 0x007FFFFF) | 0x3F800000, jnp.float32)
    big = m > 1.4142135
    m = jnp.where(big, 0.5 * m, m)
    e = e + jnp.where(big, 1, 0)
    t = (m - 1.0) / (m + 1.0)
    t2 = t * t
    p = 2.0 * t * (1.0 + t2 * (1.0 / 3.0 + t2 * (0.2 + t2 * (1.0 / 7.0 + t2 * (1.0 / 9.0)))))
    return e.astype(jnp.float32) * 0.6931471805599453 + p


def _body(p_hbm, vc_hbm, out_hbm, pv, vcv, btab, ytab, wtab, sem):
    cps = [pltpu.async_copy(p_hbm, pv, sem),
           pltpu.async_copy(vc_hbm, vcv, sem)]
    for c in cps:
        c.wait()

    lane = lax.iota(jnp.int32, 16)
    mrow = lane // 3
    vcomp = lane - 3 * mrow
    in12 = lane < 12
    rot1 = jnp.where(in12, 3 * mrow + (vcomp + 1) % 3, lane)
    rot2 = jnp.where(in12, 3 * mrow + (vcomp + 2) % 3, lane)
    safe_lane = jnp.where(in12, lane, 0)
    safe_row = jnp.where(in12, mrow, 0)

    def wfill(i, carry):
        a = i // 4
        j = i - 4 * a
        wtab[pl.ds(i * 16, 16)] = plsc.load_gather(
            pv, [_WH + a * 16 + safe_row * 4 + j])
        wtab[pl.ds((12 + i) * 16, 16)] = plsc.load_gather(
            pv, [_WV + a * 16 + safe_row * 4 + j])
        return carry

    lax.fori_loop(0, 12, wfill, 0)

    def bfill(a, carry):
        wtab[pl.ds((24 + a) * 16, 16)] = plsc.load_gather(
            pv, [_BH + a * 4 + safe_row])
        wtab[pl.ds((27 + a) * 16, 16)] = plsc.load_gather(
            pv, [_BV + a * 4 + safe_row])
        return carry

    lax.fori_loop(0, 3, bfill, 0)

    def bfold(i, carry):
        wregs = [plsc.load_gather(pv, [_WS + i * 256 + lane * 16 + r])
                 for r in range(16)]
        vv = [vcv[pl.ds(r * 256 + i * 16, 16)] for r in range(12)]
        for h in range(4):
            for v in range(3):
                acc = vv[v] * wregs[h]
                for b in range(1, 4):
                    acc = acc + vv[b * 3 + v] * wregs[b * 4 + h]
                btab[pl.ds((h * 3 + v) * 256 + i * 16, 16)] = acc
        return carry

    lax.fori_loop(0, 16, bfold, 0)

    def local_update(base, wbase, bbase, t):
        tot = None
        for a, bit in enumerate((4, 2, 1)):
            nb = base + (t ^ bit) * 12
            prev = plsc.load_gather(pv, [nb + safe_lane])
            sq = prev * prev
            ln2 = sq + _vtake(sq, rot1) + _vtake(sq, rot2)
            inv = _rsqrt(ln2)
            lng = ln2 * inv
            acc = wtab[pl.ds((bbase + a) * 16, 16)]
            for j in range(4):
                acc = acc + (wtab[pl.ds((wbase + a * 4 + j) * 16, 16)]
                             * _vtake(lng, _splat_i(j * 3)))
            xe = jnp.where(acc > 0, acc, jnp.exp(acc) - 1.0)
            contrib = xe * prev * inv
            tot = contrib if tot is None else tot + contrib
        return tot

    lp = pv[pl.ds(_LP, 16)]
    conf_f = pv[pl.ds(_CONF, 16)]

    def step(t, lp):
        loc = (local_update(_HID, 0, 24, t)
               + local_update(_VIS, 12, 27, t))
        plsc.store_scatter(pv, [_HID + t * 12 + safe_lane], loc, mask=in12)
        lb = [_vtake(loc, _splat_i(d)) for d in range(12)]

        def ybody(i, mv):
            acc = btab[pl.ds(i * 16, 16)] * lb[0]
            for d in range(1, 12):
                acc = acc + btab[pl.ds(d * 256 + i * 16, 16)] * lb[d]
            ytab[pl.ds(i * 16, 16)] = acc
            return jnp.maximum(mv, acc)

        mvec = lax.fori_loop(0, 16, ybody,
                             jnp.full((16,), -jnp.inf, jnp.float32))
        msp = lax.broadcast(jnp.max(mvec), (16,))

        def sbody(i, sv):
            return sv + jnp.exp(ytab[pl.ds(i * 16, 16)] - msp)

        svec = lax.fori_loop(0, 16, sbody, jnp.zeros((16,), jnp.float32))
        logs = _vlog(lax.broadcast(jnp.sum(svec), (16,)))

        cvec = plsc.bitcast(_vtake(conf_f, lax.broadcast(t, (16,))),
                            jnp.int32)
        ycv = plsc.load_gather(ytab, [cvec])
        lp = lp + 0.5 * (ycv - msp - logs)

        vis_new = plsc.load_gather(vcv, [3072 + cvec * 12 + safe_lane])
        plsc.store_scatter(pv, [_VIS + t * 12 + safe_lane], vis_new,
                           mask=in12)
        return lp

    lp = lax.fori_loop(0, 8, step, lp)

    pv[pl.ds(_LP, 16)] = lp
    pltpu.sync_copy(pv.at[pl.ds(0, 208)], out_hbm)


@functools.cache
def _sc_call():
    f32 = jnp.float32
    return pl.kernel(
        _body,
        out_type=jax.ShapeDtypeStruct((208,), f32),
        mesh=plsc.VectorSubcoreMesh(core_axis_name="c", subcore_axis_name="s",
                                    num_cores=1, num_subcores=1),
        compiler_params=pltpu.CompilerParams(needs_layout_passes=False),
        scratch_types=[
            pltpu.VMEM((_PLEN,), f32),
            pltpu.VMEM((6144,), f32),
            pltpu.VMEM((3072,), f32),
            pltpu.VMEM((256,), f32),
            pltpu.VMEM((480,), f32),
            pltpu.SemaphoreType.DMA,
        ],
    )


def kernel(key, hidden_state, visible_state, logpsi, linear_conf, n,
           W_h, W_v, b_h, b_v, W_s):
    del key, n
    packed = jnp.concatenate([
        hidden_state.reshape(96), visible_state.reshape(96),
        logpsi.reshape(1), jnp.zeros((15,), jnp.float32),
        W_h.reshape(48), W_v.reshape(48), b_h.reshape(12), b_v.reshape(12),
        lax.bitcast_convert_type(linear_conf, jnp.float32),
        jnp.zeros((8,), jnp.float32),
        W_s.reshape(4096),
    ])
    out = _sc_call()(packed, jnp.asarray(_VCONST))
    return (out[0:96].reshape(2, 2, 2, 4, 3),
            out[96:192].reshape(2, 2, 2, 4, 3),
            out[192], linear_conf)


# speedup vs baseline: 5.4758x; 1.0021x over previous
import functools
from itertools import product

import numpy as np
import jax
import jax.numpy as jnp
from jax import lax
from jax.experimental import pallas as pl
from jax.experimental.pallas import tpu as pltpu
from jax.experimental.pallas import tpu_sc as plsc

_tetra = np.array(
    [[0.0, 0.0, 1.0],
     [(8.0 / 9.0) ** 0.5, 0.0, -1.0 / 3.0],
     [-(2.0 / 9.0) ** 0.5, (2.0 / 3.0) ** 0.5, -1.0 / 3.0],
     [-(2.0 / 9.0) ** 0.5, -(2.0 / 3.0) ** 0.5, -1.0 / 3.0]],
    dtype=np.float32)
_confs = np.array(list(product(range(4), repeat=4)), dtype=np.int32)
_vecs = _tetra[_confs]
_VCONST = np.concatenate([
    _vecs.transpose(1, 2, 0).reshape(-1),
    _vecs.reshape(-1),
])

_HID, _VIS, _LP, _WH, _WV, _BH, _BV, _CONF, _WS = (
    0, 96, 192, 208, 256, 304, 316, 328, 344)
_PLEN = 344 + 4096

_GDN = lax.GatherDimensionNumbers(
    offset_dims=(), collapsed_slice_dims=(0,), start_index_map=(0,))


def _vtake(x, idx):
    return lax.gather(x, idx[:, None], _GDN, (1,),
                      mode=lax.GatherScatterMode.PROMISE_IN_BOUNDS)


def _splat_i(v):
    return lax.broadcast(jnp.asarray(v, jnp.int32), (16,))


def _rsqrt(x):
    b = plsc.bitcast(x, jnp.int32)
    y = plsc.bitcast(jnp.int32(0x5F3759DF) - (b >> 1), jnp.float32)
    for _ in range(3):
        y = y * (1.5 - 0.5 * x * y * y)
    return y


def _vlog(x):
    b = plsc.bitcast(x, jnp.int32)
    e = ((b >> 23) & 0xFF) - 127
    m = plsc.bitcast((b & 0x007FFFFF) | 0x3F800000, jnp.float32)
    big = m > 1.4142135
    m = jnp.where(big, 0.5 * m, m)
    e = e + jnp.where(big, 1, 0)
    t = (m - 1.0) / (m + 1.0)
    t2 = t * t
    p = 2.0 * t * (1.0 + t2 * (1.0 / 3.0 + t2 * (0.2 + t2 * (1.0 / 7.0 + t2 * (1.0 / 9.0)))))
    return e.astype(jnp.float32) * 0.6931471805599453 + p


def _body(p_hbm, vc_hbm, out_hbm, pv, vcv, btab, ytab, wtab, sem):
    cps = [pltpu.async_copy(p_hbm, pv, sem),
           pltpu.async_copy(vc_hbm, vcv, sem)]
    for c in cps:
        c.wait()

    lane = lax.iota(jnp.int32, 16)
    mrow = lane // 3
    vcomp = lane - 3 * mrow
    in12 = lane < 12
    rot1 = jnp.where(in12, 3 * mrow + (vcomp + 1) % 3, lane)
    rot2 = jnp.where(in12, 3 * mrow + (vcomp + 2) % 3, lane)
    safe_lane = jnp.where(in12, lane, 0)
    safe_row = jnp.where(in12, mrow, 0)

    def wfill(i, carry):
        a = i // 4
        j = i - 4 * a
        wtab[pl.ds(i * 16, 16)] = plsc.load_gather(
            pv, [_WH + a * 16 + safe_row * 4 + j])
        wtab[pl.ds((12 + i) * 16, 16)] = plsc.load_gather(
            pv, [_WV + a * 16 + safe_row * 4 + j])
        return carry

    lax.fori_loop(0, 12, wfill, 0)

    def bfill(a, carry):
        wtab[pl.ds((24 + a) * 16, 16)] = plsc.load_gather(
            pv, [_BH + a * 4 + safe_row])
        wtab[pl.ds((27 + a) * 16, 16)] = plsc.load_gather(
            pv, [_BV + a * 4 + safe_row])
        return carry

    lax.fori_loop(0, 3, bfill, 0)

    def bfold(i, carry):
        wregs = [plsc.load_gather(pv, [_WS + i * 256 + lane * 16 + r])
                 for r in range(16)]
        vv = [vcv[pl.ds(r * 256 + i * 16, 16)] for r in range(12)]
        for h in range(4):
            for v in range(3):
                acc = vv[v] * wregs[h]
                for b in range(1, 4):
                    acc = acc + vv[b * 3 + v] * wregs[b * 4 + h]
                btab[pl.ds((h * 3 + v) * 256 + i * 16, 16)] = acc
        return carry

    lax.fori_loop(0, 16, bfold, 0)

    lp = pv[pl.ds(_LP, 16)]
    conf_f = pv[pl.ds(_CONF, 16)]

    def step(t, lp):
        def sa_body(sa, tot):
            s = sa // 3
            a = sa - 3 * s
            bit = 4 >> a
            nb = s * 96 + (t ^ bit) * 12
            prev = plsc.load_gather(pv, [nb + safe_lane])
            sq = prev * prev
            ln2 = sq + _vtake(sq, rot1) + _vtake(sq, rot2)
            inv = _rsqrt(ln2)
            lng = ln2 * inv
            acc = wtab[pl.ds((24 + s * 3 + a) * 16, 16)]

            def jbody(j, acc):
                return acc + (wtab[pl.ds((s * 12 + a * 4 + j) * 16, 16)]
                              * _vtake(lng, lax.broadcast(j * 3, (16,))))

            acc = lax.fori_loop(0, 4, jbody, acc)
            xe = jnp.where(acc > 0, acc, jnp.exp(acc) - 1.0)
            return tot + xe * prev * inv

        loc = lax.fori_loop(0, 6, sa_body, jnp.zeros((16,), jnp.float32))
        plsc.store_scatter(pv, [_HID + t * 12 + safe_lane], loc, mask=in12)
        lb = [_vtake(loc, _splat_i(d)) for d in range(12)]

        def ybody(i, mv):
            acc = btab[pl.ds(i * 16, 16)] * lb[0]
            for d in range(1, 12):
                acc = acc + btab[pl.ds(d * 256 + i * 16, 16)] * lb[d]
            ytab[pl.ds(i * 16, 16)] = acc
            return jnp.maximum(mv, acc)

        mvec = lax.fori_loop(0, 16, ybody,
                             jnp.full((16,), -jnp.inf, jnp.float32))
        msp = lax.broadcast(jnp.max(mvec), (16,))

        def sbody(i, sv):
            return sv + jnp.exp(ytab[pl.ds(i * 16, 16)] - msp)

        svec = lax.fori_loop(0, 16, sbody, jnp.zeros((16,), jnp.float32))
        logs = _vlog(lax.broadcast(jnp.sum(svec), (16,)))

        cvec = plsc.bitcast(_vtake(conf_f, lax.broadcast(t, (16,))),
                            jnp.int32)
        ycv = plsc.load_gather(ytab, [cvec])
        lp = lp + 0.5 * (ycv - msp - logs)

        vis_new = plsc.load_gather(vcv, [3072 + cvec * 12 + safe_lane])
        plsc.store_scatter(pv, [_VIS + t * 12 + safe_lane], vis_new,
                           mask=in12)
        return lp

    lp = lax.fori_loop(0, 8, step, lp)

    pv[pl.ds(_LP, 16)] = lp
    pltpu.sync_copy(pv.at[pl.ds(0, 208)], out_hbm)


@functools.cache
def _sc_call():
    f32 = jnp.float32
    return pl.kernel(
        _body,
        out_type=jax.ShapeDtypeStruct((208,), f32),
        mesh=plsc.VectorSubcoreMesh(core_axis_name="c", subcore_axis_name="s",
                                    num_cores=1, num_subcores=1),
        compiler_params=pltpu.CompilerParams(needs_layout_passes=False),
        scratch_types=[
            pltpu.VMEM((_PLEN,), f32),
            pltpu.VMEM((6144,), f32),
            pltpu.VMEM((3072,), f32),
            pltpu.VMEM((256,), f32),
            pltpu.VMEM((480,), f32),
            pltpu.SemaphoreType.DMA,
        ],
    )


def kernel(key, hidden_state, visible_state, logpsi, linear_conf, n,
           W_h, W_v, b_h, b_v, W_s):
    del key, n
    packed = jnp.concatenate([
        hidden_state.reshape(96), visible_state.reshape(96),
        logpsi.reshape(1), jnp.zeros((15,), jnp.float32),
        W_h.reshape(48), W_v.reshape(48), b_h.reshape(12), b_v.reshape(12),
        lax.bitcast_convert_type(linear_conf, jnp.float32),
        jnp.zeros((8,), jnp.float32),
        W_s.reshape(4096),
    ])
    out = _sc_call()(packed, jnp.asarray(_VCONST))
    return (out[0:96].reshape(2, 2, 2, 4, 3),
            out[96:192].reshape(2, 2, 2, 4, 3),
            out[192], linear_conf)


# speedup vs baseline: 5.5298x; 1.0099x over previous
import functools
from itertools import product

import numpy as np
import jax
import jax.numpy as jnp
from jax import lax
from jax.experimental import pallas as pl
from jax.experimental.pallas import tpu as pltpu
from jax.experimental.pallas import tpu_sc as plsc

_tetra = np.array(
    [[0.0, 0.0, 1.0],
     [(8.0 / 9.0) ** 0.5, 0.0, -1.0 / 3.0],
     [-(2.0 / 9.0) ** 0.5, (2.0 / 3.0) ** 0.5, -1.0 / 3.0],
     [-(2.0 / 9.0) ** 0.5, -(2.0 / 3.0) ** 0.5, -1.0 / 3.0]],
    dtype=np.float32)
_confs = np.array(list(product(range(4), repeat=4)), dtype=np.int32)
_vecs = _tetra[_confs]
_VCONST = np.concatenate([
    _vecs.transpose(1, 2, 0).reshape(-1),
    _vecs.reshape(-1),
])

_HID, _VIS, _LP, _WH, _WV, _BH, _BV, _CONF, _WS = (
    0, 96, 192, 208, 256, 304, 316, 328, 344)
_PLEN = 344 + 4096

_GDN = lax.GatherDimensionNumbers(
    offset_dims=(), collapsed_slice_dims=(0,), start_index_map=(0,))


def _vtake(x, idx):
    return lax.gather(x, idx[:, None], _GDN, (1,),
                      mode=lax.GatherScatterMode.PROMISE_IN_BOUNDS)


def _splat_i(v):
    return lax.broadcast(jnp.asarray(v, jnp.int32), (16,))


def _rsqrt(x):
    b = plsc.bitcast(x, jnp.int32)
    y = plsc.bitcast(jnp.int32(0x5F3759DF) - (b >> 1), jnp.float32)
    for _ in range(3):
        y = y * (1.5 - 0.5 * x * y * y)
    return y


def _vlog(x):
    b = plsc.bitcast(x, jnp.int32)
    e = ((b >> 23) & 0xFF) - 127
    m = plsc.bitcast((b & 0x007FFFFF) | 0x3F800000, jnp.float32)
    big = m > 1.4142135
    m = jnp.where(big, 0.5 * m, m)
    e = e + jnp.where(big, 1, 0)
    t = (m - 1.0) / (m + 1.0)
    t2 = t * t
    p = 2.0 * t * (1.0 + t2 * (1.0 / 3.0 + t2 * (0.2 + t2 * (1.0 / 7.0 + t2 * (1.0 / 9.0)))))
    return e.astype(jnp.float32) * 0.6931471805599453 + p


def _body(p_hbm, vc_hbm, out_hbm, pv, vcv, btab, ytab, wtab, sem):
    cps = [pltpu.async_copy(p_hbm, pv, sem),
           pltpu.async_copy(vc_hbm, vcv, sem)]
    for c in cps:
        c.wait()

    lane = lax.iota(jnp.int32, 16)
    mrow = lane // 3
    vcomp = lane - 3 * mrow
    in12 = lane < 12
    rot1 = jnp.where(in12, 3 * mrow + (vcomp + 1) % 3, lane)
    rot2 = jnp.where(in12, 3 * mrow + (vcomp + 2) % 3, lane)
    safe_lane = jnp.where(in12, lane, 0)
    safe_row = jnp.where(in12, mrow, 0)

    def wfill(i, carry):
        a = i // 4
        j = i - 4 * a
        wtab[pl.ds(i * 16, 16)] = plsc.load_gather(
            pv, [_WH + a * 16 + safe_row * 4 + j])
        wtab[pl.ds((12 + i) * 16, 16)] = plsc.load_gather(
            pv, [_WV + a * 16 + safe_row * 4 + j])
        return carry

    lax.fori_loop(0, 12, wfill, 0)

    def bfill(a, carry):
        wtab[pl.ds((24 + a) * 16, 16)] = plsc.load_gather(
            pv, [_BH + a * 4 + safe_row])
        wtab[pl.ds((27 + a) * 16, 16)] = plsc.load_gather(
            pv, [_BV + a * 4 + safe_row])
        return carry

    lax.fori_loop(0, 3, bfill, 0)

    def bfold(i, carry):
        wregs = [plsc.load_gather(pv, [_WS + i * 256 + lane * 16 + r])
                 for r in range(16)]
        vv = [vcv[pl.ds(r * 256 + i * 16, 16)] for r in range(12)]
        for h in range(4):
            for v in range(3):
                acc = vv[v] * wregs[h]
                for b in range(1, 4):
                    acc = acc + vv[b * 3 + v] * wregs[b * 4 + h]
                btab[pl.ds((h * 3 + v) * 256 + i * 16, 16)] = acc
        return carry

    lax.fori_loop(0, 16, bfold, 0)

    def local_update(base, wbase, bbase, t):
        tot = None
        for a, bit in enumerate((4, 2, 1)):
            nb = base + (t ^ bit) * 12
            prev = plsc.load_gather(pv, [nb + safe_lane])
            sq = prev * prev
            ln2 = sq + _vtake(sq, rot1) + _vtake(sq, rot2)
            inv = _rsqrt(ln2)
            lng = ln2 * inv
            acc = wtab[pl.ds((bbase + a) * 16, 16)]
            for j in range(4):
                acc = acc + (wtab[pl.ds((wbase + a * 4 + j) * 16, 16)]
                             * _vtake(lng, _splat_i(j * 3)))
            xe = jnp.where(acc > 0, acc, jnp.exp(acc) - 1.0)
            contrib = xe * prev * inv
            tot = contrib if tot is None else tot + contrib
        return tot

    lp = pv[pl.ds(_LP, 16)]
    conf_f = pv[pl.ds(_CONF, 16)]

    def step(t, lp):
        loc = (local_update(_HID, 0, 24, t)
               + local_update(_VIS, 12, 27, t))
        plsc.store_scatter(pv, [_HID + t * 12 + safe_lane], loc, mask=in12)
        lb = [_vtake(loc, _splat_i(d)) for d in range(12)]

        def ybody(i, mv):
            acc = btab[pl.ds(i * 16, 16)] * lb[0]
            for d in range(1, 12):
                acc = acc + btab[pl.ds(d * 256 + i * 16, 16)] * lb[d]
            ytab[pl.ds(i * 16, 16)] = acc
            return jnp.maximum(mv, acc)

        mvec = lax.fori_loop(0, 16, ybody,
                             jnp.full((16,), -jnp.inf, jnp.float32))
        msp = lax.broadcast(jnp.max(mvec), (16,))

        def sbody(i, sv):
            return sv + jnp.exp(ytab[pl.ds(i * 16, 16)] - msp)

        svec = lax.fori_loop(0, 16, sbody, jnp.zeros((16,), jnp.float32))
        logs = _vlog(lax.broadcast(jnp.sum(svec), (16,)))

        cvec = plsc.bitcast(_vtake(conf_f, lax.broadcast(t, (16,))),
                            jnp.int32)
        ycv = plsc.load_gather(ytab, [cvec])
        lp = lp + 0.5 * (ycv - msp - logs)

        vis_new = plsc.load_gather(vcv, [3072 + cvec * 12 + safe_lane])
        plsc.store_scatter(pv, [_VIS + t * 12 + safe_lane], vis_new,
                           mask=in12)
        return lp

    lp = lax.fori_loop(0, 8, step, lp)

    pv[pl.ds(_LP, 16)] = lp
    pltpu.sync_copy(pv.at[pl.ds(0, 208)], out_hbm)


@functools.cache
def _sc_call():
    f32 = jnp.float32
    return pl.kernel(
        _body,
        out_type=jax.ShapeDtypeStruct((208,), f32),
        mesh=plsc.VectorSubcoreMesh(core_axis_name="c", subcore_axis_name="s",
                                    num_cores=1, num_subcores=1),
        compiler_params=pltpu.CompilerParams(needs_layout_passes=False),
        scratch_types=[
            pltpu.VMEM((_PLEN,), f32),
            pltpu.VMEM((6144,), f32),
            pltpu.VMEM((3072,), f32),
            pltpu.VMEM((256,), f32),
            pltpu.VMEM((480,), f32),
            pltpu.SemaphoreType.DMA,
        ],
    )


def kernel(key, hidden_state, visible_state, logpsi, linear_conf, n,
           W_h, W_v, b_h, b_v, W_s):
    del key, n
    packed = jnp.concatenate([
        hidden_state.reshape(96), visible_state.reshape(96),
        logpsi.reshape(1), jnp.zeros((15,), jnp.float32),
        W_h.reshape(48), W_v.reshape(48), b_h.reshape(12), b_v.reshape(12),
        lax.bitcast_convert_type(linear_conf, jnp.float32),
        jnp.zeros((8,), jnp.float32),
        W_s.reshape(4096),
    ])
    out = _sc_call()(packed, jnp.asarray(_VCONST))
    return (out[0:96].reshape(2, 2, 2, 4, 3),
            out[96:192].reshape(2, 2, 2, 4, 3),
            out[192], linear_conf)
